# baseline (speedup 1.0000x reference)
import dataclasses
import functools

import jax
import jax.numpy as jnp
from jax import lax
from jax.experimental import pallas as pl
from jax.experimental.pallas import tpu as pltpu
from jax.experimental.pallas import tpu_sc as plsc

S = 2048
E = 8
C = 512
D_MODEL = 1024
D_FF = 4096
NSLOT = E * C
NPAD = NSLOT + C
FBLK = 2048

@functools.cache
def _mesh():
    return plsc.VectorSubcoreMesh(core_axis_name="core", subcore_axis_name="subcore")


def _gate_body(rx_ref, wg_ref, f1_ref, f2_ref, w1_ref, w2_ref):
    rx = rx_ref[...]
    logits_se = jnp.dot(rx, wg_ref[...], preferred_element_type=jnp.float32)
    lt = jnp.transpose(logits_se, (1, 0))
    m = jnp.max(lt, axis=0, keepdims=True)
    eg = jnp.exp(lt - m)
    gates = eg / jnp.sum(eg, axis=0, keepdims=True)

    eio = lax.broadcasted_iota(jnp.int32, (E, S), 0).astype(jnp.float32)
    g1m = jnp.max(gates, axis=0, keepdims=True)
    idx1 = jnp.min(jnp.where(gates == g1m, eio, float(E)), axis=0, keepdims=True)
    mask1 = (eio == idx1).astype(jnp.float32)
    logits2 = jnp.where(mask1 > 0, -1e9, lt)
    l2m = jnp.max(logits2, axis=0, keepdims=True)
    idx2 = jnp.min(jnp.where(logits2 == l2m, eio, float(E)), axis=0, keepdims=True)
    mask2 = (eio == idx2).astype(jnp.float32)

    def cumsum_lanes(x):
        k = 1
        while k < S:
            x = x + jnp.concatenate(
                [jnp.zeros((E, k), jnp.float32), x[:, :S - k]], axis=1)
            k *= 2
        return x

    cs1 = cumsum_lanes(mask1)
    cs2 = cumsum_lanes(mask2)
    count1 = cs1[:, S - 1:S]
    locations1 = cs1 - 1.0
    locations2 = cs2 - 1.0 + count1

    mask1 = mask1 * (locations1 < float(C)).astype(jnp.float32)
    mask2 = mask2 * (locations2 < float(C)).astype(jnp.float32)
    loc1 = jnp.sum(locations1 * mask1, axis=0, keepdims=True)
    loc2 = jnp.sum(locations2 * mask2, axis=0, keepdims=True)

    g1 = jnp.sum(gates * mask1, axis=0, keepdims=True)
    g2 = jnp.sum(gates * mask2, axis=0, keepdims=True)
    denom = g1 + g2
    denom = jnp.where(denom < 1e-9, 1.0, denom)
    w1 = g1 / denom
    w2 = g2 / denom

    k1 = jnp.sum(mask1, axis=0, keepdims=True) > 0.0
    k2 = jnp.sum(mask2, axis=0, keepdims=True) > 0.0
    f1 = jnp.where(k1, idx1 * float(C) + loc1, float(NSLOT))
    f2 = jnp.where(k2, idx2 * float(C) + loc2, float(NSLOT))

    f1_ref[...] = f1.astype(jnp.int32)
    f2_ref[...] = f2.astype(jnp.int32)
    w1_ref[...] = w1
    w2_ref[...] = w2


def _gate(rx, Wg):
    return pl.pallas_call(
        _gate_body,
        out_shape=[
            jax.ShapeDtypeStruct((1, S), jnp.int32),
            jax.ShapeDtypeStruct((1, S), jnp.int32),
            jax.ShapeDtypeStruct((1, S), jnp.float32),
            jax.ShapeDtypeStruct((1, S), jnp.float32),
        ],
    )(rx, Wg)


def _sc_compiler_params():
    cp = pltpu.CompilerParams()
    if "needs_layout_passes" in pltpu.CompilerParams.__dataclass_fields__:
        cp = dataclasses.replace(cp, needs_layout_passes=False)
    return cp


_DW = 32


@functools.cache
def _dispatch_kernel(eg):
    half = NSLOT // 2
    n_per_w = half // 32
    n_chunks = n_per_w // _DW

    @functools.partial(
        pl.kernel,
        mesh=_mesh(),
        compiler_params=_sc_compiler_params(),
        out_type=jax.ShapeDtypeStruct((NSLOT // 2, D_MODEL), jnp.float32),
        scratch_types=[
            pltpu.VMEM((S,), jnp.int32),
            pltpu.VMEM((S,), jnp.int32),
            pltpu.VMEM((NSLOT,), jnp.int32),
            pltpu.VMEM((_DW, D_MODEL), jnp.float32),
            pltpu.VMEM((_DW, D_MODEL), jnp.float32),
            pltpu.SemaphoreType.DMA,
            pltpu.SemaphoreType.DMA,
            pltpu.SemaphoreType.DMA,
            pltpu.SemaphoreType.DMA,
        ],
    )
    def _dispatch(x_hbm, f1_hbm, f2_hbm, out_hbm,
                  f1_v, f2_v, st_v, rows_a, rows_b,
                  gsem_a, gsem_b, ssem_a, ssem_b):
        wid = lax.axis_index("subcore") * 2 + lax.axis_index("core")
        base = wid * n_per_w
        slot_base = eg * half + base
        pltpu.sync_copy(f1_hbm.at[0], f1_v)
        pltpu.sync_copy(f2_hbm.at[0], f2_v)

        @pl.loop(0, n_per_w // 16)
        def _(i):
            st_v[pl.ds(slot_base + i * 16, 16)] = jnp.zeros((16,), jnp.int32)

        for f_v in (f1_v, f2_v):
            @pl.loop(0, S // 32)
            def _(i):
                sl0 = pl.ds(i * 32, 16)
                sl1 = pl.ds(i * 32 + 16, 16)
                idx0 = f_v[sl0]
                idx1 = f_v[sl1]
                lane = lax.broadcasted_iota(jnp.int32, (16,), 0)
                tok0 = lane + i * 32
                tok1 = lane + (i * 32 + 16)
                m0 = idx0 < NSLOT
                m1 = idx1 < NSLOT
                plsc.store_scatter(st_v, [jnp.where(m0, idx0, 0)], tok0, mask=m0)
                plsc.store_scatter(st_v, [jnp.where(m1, idx1, 0)], tok1, mask=m1)

        bufs = (rows_a, rows_b)
        gsems = (gsem_a, gsem_b)
        ssems = (ssem_a, ssem_b)
        stores = [None, None]
        prev = None
        for ch in range(n_chunks):
            b = ch % 2
            if stores[b] is not None:
                stores[b].wait()
                stores[b] = None
            cp = pltpu.async_copy(
                x_hbm.at[st_v.at[pl.ds(slot_base + ch * _DW, _DW)]],
                bufs[b], gsems[b])
            if prev is not None:
                pch, pcp = prev
                pb = pch % 2
                pcp.wait()
                stores[pb] = pltpu.async_copy(
                    bufs[pb], out_hbm.at[pl.ds(base + pch * _DW, _DW)], ssems[pb])
            prev = (ch, cp)
        pch, pcp = prev
        pb = pch % 2
        pcp.wait()
        pltpu.sync_copy(bufs[pb], out_hbm.at[pl.ds(base + pch * _DW, _DW)])
        for st in stores:
            if st is not None:
                st.wait()

    return _dispatch


def _ffn_body(disp_ref, w1_ref, w2_ref, *rest):
    out_ref, acc_ref = rest[-2], rest[-1]
    f = pl.program_id(1)

    @pl.when(f == 0)
    def _():
        acc_ref[...] = jnp.zeros_like(acc_ref)

    h = jnp.maximum(
        jnp.dot(disp_ref[...], w1_ref[0], preferred_element_type=jnp.float32),
        0.0)
    acc_ref[...] += jnp.dot(h, w2_ref[0], preferred_element_type=jnp.float32)

    @pl.when(f == D_FF // FBLK - 1)
    def _():
        out_ref[...] = acc_ref[...]


def _ffn_half(disp_half, W1, W2, eg, eo_prev=None):
    grid = (E // 2, D_FF // FBLK)
    e0 = eg * (E // 2)
    in_specs = [
        pl.BlockSpec((C, D_MODEL), lambda e, f: (e, 0)),
        pl.BlockSpec((1, D_MODEL, FBLK), lambda e, f: (e0 + e, 0, f)),
        pl.BlockSpec((1, FBLK, D_MODEL), lambda e, f: (e0 + e, f, 0)),
    ]
    args = [disp_half, W1, W2]
    aliases = {}
    if eo_prev is not None:
        in_specs.append(pl.BlockSpec(memory_space=pl.ANY))
        args.append(eo_prev)
        aliases = {3: 0}
    return pl.pallas_call(
        _ffn_body,
        grid=grid,
        in_specs=in_specs,
        out_specs=pl.BlockSpec((C, D_MODEL), lambda e, f: (e0 + e, 0)),
        out_shape=jax.ShapeDtypeStruct((NSLOT, D_MODEL), jnp.float32),
        scratch_shapes=[pltpu.VMEM((C, D_MODEL), jnp.float32)],
        input_output_aliases=aliases,
        compiler_params=pltpu.CompilerParams(
            dimension_semantics=("parallel", "arbitrary")),
    )(*args)


_CW = 16


@functools.cache
def _combine_kernel():
    n_chunks = S // 32 // _CW

    @functools.partial(
        pl.kernel,
        mesh=_mesh(),
        compiler_params=_sc_compiler_params(),
        out_type=jax.ShapeDtypeStruct((S, D_MODEL), jnp.float32),
        scratch_types=[
            pltpu.VMEM((S // 32,), jnp.int32),
            pltpu.VMEM((S // 32,), jnp.int32),
            pltpu.VMEM((S // 32,), jnp.float32),
            pltpu.VMEM((S // 32,), jnp.float32),
            pltpu.VMEM((_CW, D_MODEL), jnp.float32),
            pltpu.VMEM((_CW, D_MODEL), jnp.float32),
            pltpu.VMEM((_CW, D_MODEL), jnp.float32),
            pltpu.VMEM((_CW, D_MODEL), jnp.float32),
            pltpu.SemaphoreType.DMA,
            pltpu.SemaphoreType.DMA,
            pltpu.SemaphoreType.DMA,
            pltpu.SemaphoreType.DMA,
        ],
    )
    def _combine(eo_hbm, i1_hbm, i2_hbm, w1_hbm, w2_hbm, out_hbm,
                 i1_v, i2_v, w1_v, w2_v, r1a_v, r2a_v, r1b_v, r2b_v,
                 sem_a1, sem_a2, sem_b1, sem_b2):
        wid = lax.axis_index("subcore") * 2 + lax.axis_index("core")
        n_per_w = S // 32
        base = wid * n_per_w
        pltpu.sync_copy(i1_hbm.at[0, pl.ds(base, n_per_w)], i1_v)
        pltpu.sync_copy(i2_hbm.at[0, pl.ds(base, n_per_w)], i2_v)
        pltpu.sync_copy(w1_hbm.at[0, pl.ds(base, n_per_w)], w1_v)
        pltpu.sync_copy(w2_hbm.at[0, pl.ds(base, n_per_w)], w2_v)

        @pl.loop(0, n_per_w // 16)
        def _(i):
            sl = pl.ds(i * 16, 16)
            i1_v[sl] = jnp.minimum(i1_v[sl], NSLOT - 1)
            i2_v[sl] = jnp.minimum(i2_v[sl], NSLOT - 1)

        pairs = ((r1a_v, r2a_v, sem_a1, sem_a2), (r1b_v, r2b_v, sem_b1, sem_b2))

        def fire(ch):
            r1_v, r2_v, s1, s2 = pairs[ch % 2]
            a = pltpu.async_copy(
                eo_hbm.at[i1_v.at[pl.ds(ch * _CW, _CW)]], r1_v, s1)
            b = pltpu.async_copy(
                eo_hbm.at[i2_v.at[pl.ds(ch * _CW, _CW)]], r2_v, s2)
            return a, b

        def drain(ch, cps):
            r1_v, r2_v, _, _ = pairs[ch % 2]
            cps[0].wait()
            cps[1].wait()

            @pl.loop(0, _CW)
            def _(r):
                t = ch * _CW + r
                wa = plsc.load_gather(w1_v, [jnp.full((16,), t, jnp.int32)])
                wb = plsc.load_gather(w2_v, [jnp.full((16,), t, jnp.int32)])

                @pl.loop(0, D_MODEL, step=128)
                def _(c):
                    sls = [pl.ds(c + u * 16, 16) for u in range(8)]
                    avs = [r1_v[r, sl] for sl in sls]
                    bvs = [r2_v[r, sl] for sl in sls]
                    for u in range(8):
                        r1_v[r, sls[u]] = wa * avs[u] + wb * bvs[u]

            pltpu.sync_copy(r1_v, out_hbm.at[pl.ds(base + ch * _CW, _CW)])

        prev = None
        for ch in range(n_chunks):
            cps = fire(ch)
            if prev is not None:
                drain(prev[0], prev[1])
            prev = (ch, cps)
        drain(prev[0], prev[1])

    return _combine


def kernel(x, Wg, W1, W2):
    rx = x.reshape(S, D_MODEL)
    f1, f2, w1, w2 = _gate(rx, Wg)
    disp_a = _dispatch_kernel(0)(rx, f1, f2)
    disp_b = _dispatch_kernel(1)(rx, f1, f2)
    eo_a = _ffn_half(disp_a, W1, W2, 0)
    eo = _ffn_half(disp_b, W1, W2, 1, eo_prev=eo_a)
    y = _combine_kernel()(eo, f1, f2, w1, w2)
    return y.reshape(x.shape)


# speedup vs baseline: 1.0239x; 1.0239x over previous
import dataclasses
import functools

import jax
import jax.numpy as jnp
from jax import lax
from jax.experimental import pallas as pl
from jax.experimental.pallas import tpu as pltpu
from jax.experimental.pallas import tpu_sc as plsc

S = 2048
E = 8
C = 512
D_MODEL = 1024
D_FF = 4096
NSLOT = E * C
NPAD = NSLOT + C
FBLK = 2048

@functools.cache
def _mesh():
    return plsc.VectorSubcoreMesh(core_axis_name="core", subcore_axis_name="subcore")


def _gate_body(rx_ref, wg_ref, f1_ref, f2_ref, w1_ref, w2_ref):
    rx = rx_ref[...]
    logits_se = jnp.dot(rx, wg_ref[...], preferred_element_type=jnp.float32)
    lt = jnp.transpose(logits_se, (1, 0))
    m = jnp.max(lt, axis=0, keepdims=True)
    eg = jnp.exp(lt - m)
    gates = eg / jnp.sum(eg, axis=0, keepdims=True)

    eio = lax.broadcasted_iota(jnp.int32, (E, S), 0).astype(jnp.float32)
    g1m = jnp.max(gates, axis=0, keepdims=True)
    idx1 = jnp.min(jnp.where(gates == g1m, eio, float(E)), axis=0, keepdims=True)
    mask1 = (eio == idx1).astype(jnp.float32)
    logits2 = jnp.where(mask1 > 0, -1e9, lt)
    l2m = jnp.max(logits2, axis=0, keepdims=True)
    idx2 = jnp.min(jnp.where(logits2 == l2m, eio, float(E)), axis=0, keepdims=True)
    mask2 = (eio == idx2).astype(jnp.float32)

    def cumsum_lanes(x):
        k = 1
        while k < S:
            x = x + jnp.concatenate(
                [jnp.zeros((E, k), jnp.float32), x[:, :S - k]], axis=1)
            k *= 2
        return x

    cs1 = cumsum_lanes(mask1)
    cs2 = cumsum_lanes(mask2)
    count1 = cs1[:, S - 1:S]
    locations1 = cs1 - 1.0
    locations2 = cs2 - 1.0 + count1

    mask1 = mask1 * (locations1 < float(C)).astype(jnp.float32)
    mask2 = mask2 * (locations2 < float(C)).astype(jnp.float32)
    loc1 = jnp.sum(locations1 * mask1, axis=0, keepdims=True)
    loc2 = jnp.sum(locations2 * mask2, axis=0, keepdims=True)

    g1 = jnp.sum(gates * mask1, axis=0, keepdims=True)
    g2 = jnp.sum(gates * mask2, axis=0, keepdims=True)
    denom = g1 + g2
    denom = jnp.where(denom < 1e-9, 1.0, denom)
    w1 = g1 / denom
    w2 = g2 / denom

    k1 = jnp.sum(mask1, axis=0, keepdims=True) > 0.0
    k2 = jnp.sum(mask2, axis=0, keepdims=True) > 0.0
    f1 = jnp.where(k1, idx1 * float(C) + loc1, float(NSLOT))
    f2 = jnp.where(k2, idx2 * float(C) + loc2, float(NSLOT))

    f1_ref[...] = f1.astype(jnp.int32)
    f2_ref[...] = f2.astype(jnp.int32)
    w1_ref[...] = w1
    w2_ref[...] = w2


def _gate(rx, Wg):
    return pl.pallas_call(
        _gate_body,
        out_shape=[
            jax.ShapeDtypeStruct((1, S), jnp.int32),
            jax.ShapeDtypeStruct((1, S), jnp.int32),
            jax.ShapeDtypeStruct((1, S), jnp.float32),
            jax.ShapeDtypeStruct((1, S), jnp.float32),
        ],
    )(rx, Wg)


def _sc_compiler_params():
    cp = pltpu.CompilerParams()
    if "needs_layout_passes" in pltpu.CompilerParams.__dataclass_fields__:
        cp = dataclasses.replace(cp, needs_layout_passes=False)
    return cp


_DW = 16


@functools.cache
def _dispatch_kernel():
    n_per_w = NSLOT // 32
    n_chunks = n_per_w // _DW

    @functools.partial(
        pl.kernel,
        mesh=_mesh(),
        compiler_params=_sc_compiler_params(),
        out_type=jax.ShapeDtypeStruct((NSLOT, D_MODEL), jnp.float32),
        scratch_types=[
            pltpu.VMEM((S,), jnp.int32),
            pltpu.VMEM((S,), jnp.int32),
            pltpu.VMEM((NSLOT,), jnp.int32),
            pltpu.VMEM((_DW, D_MODEL), jnp.float32),
            pltpu.VMEM((_DW, D_MODEL), jnp.float32),
            pltpu.VMEM((_DW, D_MODEL), jnp.float32),
            pltpu.VMEM((_DW, D_MODEL), jnp.float32),
            pltpu.SemaphoreType.DMA,
            pltpu.SemaphoreType.DMA,
            pltpu.SemaphoreType.DMA,
            pltpu.SemaphoreType.DMA,
            pltpu.SemaphoreType.DMA,
            pltpu.SemaphoreType.DMA,
            pltpu.SemaphoreType.DMA,
            pltpu.SemaphoreType.DMA,
        ],
    )
    def _dispatch(x_hbm, f1_hbm, f2_hbm, out_hbm,
                  f1_v, f2_v, st_v, rows_a, rows_b, rows_c, rows_d,
                  gsem_a, gsem_b, gsem_c, gsem_d,
                  ssem_a, ssem_b, ssem_c, ssem_d):
        wid = lax.axis_index("subcore") * 2 + lax.axis_index("core")
        base = wid * n_per_w
        pltpu.sync_copy(f1_hbm.at[0], f1_v)
        pltpu.sync_copy(f2_hbm.at[0], f2_v)

        @pl.loop(0, n_per_w // 16)
        def _(i):
            st_v[pl.ds(base + i * 16, 16)] = jnp.zeros((16,), jnp.int32)

        for f_v in (f1_v, f2_v):
            @pl.loop(0, S // 32)
            def _(i):
                sl0 = pl.ds(i * 32, 16)
                sl1 = pl.ds(i * 32 + 16, 16)
                idx0 = f_v[sl0]
                idx1 = f_v[sl1]
                lane = lax.broadcasted_iota(jnp.int32, (16,), 0)
                tok0 = lane + i * 32
                tok1 = lane + (i * 32 + 16)
                m0 = idx0 < NSLOT
                m1 = idx1 < NSLOT
                plsc.store_scatter(st_v, [jnp.where(m0, idx0, 0)], tok0, mask=m0)
                plsc.store_scatter(st_v, [jnp.where(m1, idx1, 0)], tok1, mask=m1)

        bufs = (rows_a, rows_b, rows_c, rows_d)
        gsems = (gsem_a, gsem_b, gsem_c, gsem_d)
        ssems = (ssem_a, ssem_b, ssem_c, ssem_d)
        nbuf = len(bufs)
        gathers = [None] * nbuf
        stores = [None] * nbuf
        for ch in range(n_chunks + nbuf - 1):
            b = ch % nbuf
            if ch < n_chunks:
                if stores[b] is not None:
                    stores[b].wait()
                    stores[b] = None
                gathers[b] = pltpu.async_copy(
                    x_hbm.at[st_v.at[pl.ds(base + ch * _DW, _DW)]],
                    bufs[b], gsems[b])
            pch = ch - nbuf + 1
            if 0 <= pch < n_chunks:
                pb = pch % nbuf
                gathers[pb].wait()
                stores[pb] = pltpu.async_copy(
                    bufs[pb], out_hbm.at[pl.ds(base + pch * _DW, _DW)], ssems[pb])
        for st in stores:
            if st is not None:
                st.wait()

    return _dispatch


def _ffn_body(disp_ref, w1_ref, w2_ref, out_ref, acc_ref):
    f = pl.program_id(1)

    @pl.when(f == 0)
    def _():
        acc_ref[...] = jnp.zeros_like(acc_ref)

    h = jnp.maximum(
        jnp.dot(disp_ref[...], w1_ref[0], preferred_element_type=jnp.float32),
        0.0)
    acc_ref[...] += jnp.dot(h, w2_ref[0], preferred_element_type=jnp.float32)

    @pl.when(f == D_FF // FBLK - 1)
    def _():
        out_ref[...] = acc_ref[...]


def _ffn(disp, W1, W2):
    grid = (E, D_FF // FBLK)
    return pl.pallas_call(
        _ffn_body,
        grid=grid,
        in_specs=[
            pl.BlockSpec((C, D_MODEL), lambda e, f: (e, 0)),
            pl.BlockSpec((1, D_MODEL, FBLK), lambda e, f: (e, 0, f)),
            pl.BlockSpec((1, FBLK, D_MODEL), lambda e, f: (e, f, 0)),
        ],
        out_specs=pl.BlockSpec((C, D_MODEL), lambda e, f: (e, 0)),
        out_shape=jax.ShapeDtypeStruct((NSLOT, D_MODEL), jnp.float32),
        scratch_shapes=[pltpu.VMEM((C, D_MODEL), jnp.float32)],
        compiler_params=pltpu.CompilerParams(
            dimension_semantics=("parallel", "arbitrary")),
    )(disp, W1, W2)


_CW = 16


@functools.cache
def _combine_kernel():
    n_chunks = S // 32 // _CW

    @functools.partial(
        pl.kernel,
        mesh=_mesh(),
        compiler_params=_sc_compiler_params(),
        out_type=jax.ShapeDtypeStruct((S, D_MODEL), jnp.float32),
        scratch_types=[
            pltpu.VMEM((S // 32,), jnp.int32),
            pltpu.VMEM((S // 32,), jnp.int32),
            pltpu.VMEM((S // 32,), jnp.float32),
            pltpu.VMEM((S // 32,), jnp.float32),
            pltpu.VMEM((_CW, D_MODEL), jnp.float32),
            pltpu.VMEM((_CW, D_MODEL), jnp.float32),
            pltpu.VMEM((_CW, D_MODEL), jnp.float32),
            pltpu.VMEM((_CW, D_MODEL), jnp.float32),
            pltpu.SemaphoreType.DMA,
            pltpu.SemaphoreType.DMA,
            pltpu.SemaphoreType.DMA,
            pltpu.SemaphoreType.DMA,
        ],
    )
    def _combine(eo_hbm, i1_hbm, i2_hbm, w1_hbm, w2_hbm, out_hbm,
                 i1_v, i2_v, w1_v, w2_v, r1a_v, r2a_v, r1b_v, r2b_v,
                 sem_a1, sem_a2, sem_b1, sem_b2):
        wid = lax.axis_index("subcore") * 2 + lax.axis_index("core")
        n_per_w = S // 32
        base = wid * n_per_w
        pltpu.sync_copy(i1_hbm.at[0, pl.ds(base, n_per_w)], i1_v)
        pltpu.sync_copy(i2_hbm.at[0, pl.ds(base, n_per_w)], i2_v)
        pltpu.sync_copy(w1_hbm.at[0, pl.ds(base, n_per_w)], w1_v)
        pltpu.sync_copy(w2_hbm.at[0, pl.ds(base, n_per_w)], w2_v)

        @pl.loop(0, n_per_w // 16)
        def _(i):
            sl = pl.ds(i * 16, 16)
            i1_v[sl] = jnp.minimum(i1_v[sl], NSLOT - 1)
            i2_v[sl] = jnp.minimum(i2_v[sl], NSLOT - 1)

        pairs = ((r1a_v, r2a_v, sem_a1, sem_a2), (r1b_v, r2b_v, sem_b1, sem_b2))

        def fire(ch):
            r1_v, r2_v, s1, s2 = pairs[ch % 2]
            a = pltpu.async_copy(
                eo_hbm.at[i1_v.at[pl.ds(ch * _CW, _CW)]], r1_v, s1)
            b = pltpu.async_copy(
                eo_hbm.at[i2_v.at[pl.ds(ch * _CW, _CW)]], r2_v, s2)
            return a, b

        def drain(ch, cps):
            r1_v, r2_v, _, _ = pairs[ch % 2]
            cps[0].wait()
            cps[1].wait()

            @pl.loop(0, _CW)
            def _(r):
                t = ch * _CW + r
                wa = plsc.load_gather(w1_v, [jnp.full((16,), t, jnp.int32)])
                wb = plsc.load_gather(w2_v, [jnp.full((16,), t, jnp.int32)])

                @pl.loop(0, D_MODEL, step=128)
                def _(c):
                    sls = [pl.ds(c + u * 16, 16) for u in range(8)]
                    avs = [r1_v[r, sl] for sl in sls]
                    bvs = [r2_v[r, sl] for sl in sls]
                    for u in range(8):
                        r1_v[r, sls[u]] = wa * avs[u] + wb * bvs[u]

            pltpu.sync_copy(r1_v, out_hbm.at[pl.ds(base + ch * _CW, _CW)])

        prev = None
        for ch in range(n_chunks):
            cps = fire(ch)
            if prev is not None:
                drain(prev[0], prev[1])
            prev = (ch, cps)
        drain(prev[0], prev[1])

    return _combine


def kernel(x, Wg, W1, W2):
    rx = x.reshape(S, D_MODEL)
    f1, f2, w1, w2 = _gate(rx, Wg)
    disp = _dispatch_kernel()(rx, f1, f2)
    eo = _ffn(disp, W1, W2)
    y = _combine_kernel()(eo, f1, f2, w1, w2)
    return y.reshape(x.shape)
